# baseline (speedup 1.0000x reference)
import dataclasses
import functools

import jax
import jax.numpy as jnp
from jax import lax
from jax.experimental import pallas as pl
from jax.experimental.pallas import tpu as pltpu
from jax.experimental.pallas import tpu_sc as plsc

N = 10000
E = 320000
NPAD = 10240
EPAD = 327680
NW = 2560
F = 128
C = 64
NPB = NPAD // 128

_mesh = plsc.VectorSubcoreMesh(core_axis_name="c", subcore_axis_name="s")

_sc_params = pltpu.CompilerParams()
if "needs_layout_passes" in pltpu.CompilerParams.__dataclass_fields__:
    _sc_params = dataclasses.replace(_sc_params, needs_layout_passes=False)


@jax.jit
def _k1_degrees(src_flat, dst_flat, zeros1d):
    @functools.partial(
        pl.kernel,
        out_type=jax.ShapeDtypeStruct((2, 16, NPAD), jnp.float32),
        mesh=_mesh,
        scratch_types=[
            pltpu.VMEM((EPAD // 16,), jnp.int32),
            pltpu.VMEM((NPAD,), jnp.float32),
        ],
        compiler_params=_sc_params,
    )
    def k1(src_hbm, dst_hbm, z_hbm, degp_out, idx_ref, deg_ref):
        cid = lax.axis_index("c")
        sid = lax.axis_index("s")
        chunk = EPAD // 16
        base = sid * chunk
        pltpu.sync_copy(z_hbm, deg_ref)

        @pl.when(cid == 0)
        def _():
            pltpu.sync_copy(src_hbm.at[pl.ds(base, chunk)], idx_ref)

        @pl.when(cid == 1)
        def _():
            pltpu.sync_copy(dst_hbm.at[pl.ds(base, chunk)], idx_ref)

        ones = jnp.full((16,), 1.0, jnp.float32)

        @pl.loop(0, chunk, step=16, unroll=8)
        def _(i):
            iv = idx_ref[pl.ds(i, 16)]
            plsc.addupdate_scatter(deg_ref, [iv], ones)

        pltpu.sync_copy(deg_ref, degp_out.at[cid].at[sid])

    return k1(src_flat, dst_flat, zeros1d)


@jax.jit
def _k2a_norms(degp):
    def body(degp_ref, norms_ref):
        d_out = jnp.sum(degp_ref[0:16], axis=0)
        d_in = jnp.sum(degp_ref[16:32], axis=0)
        row = lax.broadcasted_iota(jnp.int32, (NPB, 128), 0)
        col = lax.broadcasted_iota(jnp.int32, (NPB, 128), 1)
        real = (row * 128 + col) < N
        ns = jnp.where(real, lax.rsqrt(jnp.maximum(d_out, 1.0)), 0.0)
        nd = jnp.where(real, lax.rsqrt(jnp.maximum(d_in, 1.0)), 0.0)
        norms_ref[0] = ns
        norms_ref[1] = nd

    return pl.pallas_call(
        body,
        out_shape=jax.ShapeDtypeStruct((2, NPB, 128), jnp.float32),
    )(degp)


@jax.jit
def _k2b_scale(x_pad, ns_col):
    def body(x_ref, ns_ref, xn_ref):
        xn_ref[...] = x_ref[...] * ns_ref[...]

    return pl.pallas_call(
        body,
        grid=(NPAD // 512,),
        in_specs=[
            pl.BlockSpec((512, F), lambda i: (i, 0)),
            pl.BlockSpec((512, 1), lambda i: (i, 0)),
        ],
        out_specs=pl.BlockSpec((512, F), lambda i: (i, 0)),
        out_shape=jax.ShapeDtypeStruct((NPAD, F), jnp.float32),
    )(x_pad, ns_col)


@jax.jit
def _k3c_cweights(src_flat, dst_flat, nd_flat, zeros1d):
    chunk = EPAD // 32

    @functools.partial(
        pl.kernel,
        out_type=jax.ShapeDtypeStruct((2, 16, NPAD), jnp.float32),
        mesh=_mesh,
        scratch_types=[
            pltpu.VMEM((chunk,), jnp.int32),
            pltpu.VMEM((chunk,), jnp.int32),
            pltpu.VMEM((NPAD,), jnp.float32),
            pltpu.VMEM((NPAD,), jnp.float32),
        ],
        compiler_params=_sc_params,
    )
    def k3c(src_hbm, dst_hbm, nd_hbm, z_hbm, c_out,
            sidx, didx, nd_ref, c_ref):
        cid = lax.axis_index("c")
        sid = lax.axis_index("s")
        base = (cid * 16 + sid) * chunk
        pltpu.sync_copy(z_hbm, c_ref)
        pltpu.sync_copy(nd_hbm, nd_ref)
        pltpu.sync_copy(src_hbm.at[pl.ds(base, chunk)], sidx)
        pltpu.sync_copy(dst_hbm.at[pl.ds(base, chunk)], didx)

        @pl.loop(0, chunk, step=16, unroll=8)
        def _(i):
            dv = didx[pl.ds(i, 16)]
            sv = sidx[pl.ds(i, 16)]
            ndv = plsc.load_gather(nd_ref, [dv])
            plsc.addupdate_scatter(c_ref, [sv], ndv)

        pltpu.sync_copy(c_ref, c_out.at[cid].at[sid])

    return k3c(src_flat, dst_flat, nd_flat, zeros1d)


@jax.jit
def _k3_message_pass(src2d, dst2d, xn, zeros2d):
    wpw = NW // 32
    half = wpw // 2

    @functools.partial(
        pl.kernel,
        out_type=jax.ShapeDtypeStruct((2, NPAD, F), jnp.float32),
        mesh=_mesh,
        scratch_types=[
            pltpu.VMEM((half, 128), jnp.int32),
            pltpu.VMEM((half, 128), jnp.int32),
            pltpu.VMEM((128, F), jnp.float32),
            pltpu.VMEM((128, F), jnp.float32),
            pltpu.SemaphoreType.DMA,
            pltpu.SemaphoreType.DMA,
            pltpu.VMEM_SHARED((NPAD, F), jnp.float32),
        ],
        compiler_params=_sc_params,
    )
    def k3(src_hbm, dst_hbm, xn_hbm, z2_hbm,
           agg_out, sidx, didx, rowsA, rowsB, semA, semB, agg_sp):
        cid = lax.axis_index("c")
        sid = lax.axis_index("s")
        wid = cid * 16 + sid
        base = wid * wpw

        pltpu.sync_copy(z2_hbm.at[pl.ds(sid * 640, 640)],
                        agg_sp.at[pl.ds(sid * 640, 640)])
        plsc.subcore_barrier()

        for h in range(2):
            pltpu.sync_copy(src_hbm.at[pl.ds(base + h * half, half)], sidx)
            pltpu.sync_copy(dst_hbm.at[pl.ds(base + h * half, half)], didx)
            pltpu.async_copy(xn_hbm.at[sidx.at[0]], rowsA, semA)

            @pl.loop(0, half, step=2)
            def _(k):
                pltpu.async_copy(xn_hbm.at[sidx.at[k + 1]], rowsB, semB)
                pltpu.make_async_copy(
                    xn_hbm.at[sidx.at[k]], rowsA, semA).wait()
                pltpu.sync_copy(rowsA, agg_sp.at[didx.at[k]], add=True)

                @pl.when(k + 2 < half)
                def _():
                    pltpu.async_copy(
                        xn_hbm.at[sidx.at[k + 2]], rowsA, semA)

                pltpu.make_async_copy(
                    xn_hbm.at[sidx.at[k + 1]], rowsB, semB).wait()
                pltpu.sync_copy(rowsB, agg_sp.at[didx.at[k + 1]], add=True)

        plsc.subcore_barrier()
        pltpu.sync_copy(agg_sp.at[pl.ds(sid * 640, 640)],
                        agg_out.at[cid].at[pl.ds(sid * 640, 640)])

    return k3(src2d, dst2d, xn, zeros2d)


@jax.jit
def _k4_final(aggp, cp, ns_col, nd_col, W1, b1, W2, b2):
    nblk = NPAD // 512

    def body(aggp_ref, cp_ref, ns_ref, nd_ref, W1_ref, b1_ref, W2_ref,
             b2_ref, out_ref, acc_ref):
        i = pl.program_id(0)
        agg = (aggp_ref[0] + aggp_ref[1]) * nd_ref[...]
        z = jnp.dot(agg, W1_ref[...], preferred_element_type=jnp.float32)
        h = jnp.maximum(z + b1_ref[...], 0.0)
        w = jnp.sum(cp_ref[...], axis=0) * ns_ref[...] * (1.0 / N)
        pb = jnp.sum(h * w, axis=0, keepdims=True)

        @pl.when(i == 0)
        def _():
            acc_ref[...] = jnp.zeros_like(acc_ref)

        acc_ref[...] += pb

        @pl.when(i == nblk - 1)
        def _():
            out_ref[...] = (
                jnp.dot(acc_ref[...], W2_ref[...],
                        preferred_element_type=jnp.float32)
                + b2_ref[...]
            )

    return pl.pallas_call(
        body,
        grid=(nblk,),
        in_specs=[
            pl.BlockSpec((2, 512, F), lambda i: (0, i, 0)),
            pl.BlockSpec((32, 512, 1), lambda i: (0, i, 0)),
            pl.BlockSpec((512, 1), lambda i: (i, 0)),
            pl.BlockSpec((512, 1), lambda i: (i, 0)),
            pl.BlockSpec((F, F), lambda i: (0, 0)),
            pl.BlockSpec((1, F), lambda i: (0, 0)),
            pl.BlockSpec((F, C), lambda i: (0, 0)),
            pl.BlockSpec((1, C), lambda i: (0, 0)),
        ],
        out_specs=pl.BlockSpec((1, C), lambda i: (0, 0)),
        out_shape=jax.ShapeDtypeStruct((1, C), jnp.float32),
        scratch_shapes=[pltpu.VMEM((1, F), jnp.float32)],
    )(aggp, cp, ns_col, nd_col, W1, b1, W2, b2)


def kernel(in_feat, edge_index, W1, b1, W2, b2):
    pad_ids = (N + (jnp.arange(EPAD - E, dtype=jnp.int32) % (NPAD - N)))
    src = jnp.concatenate([edge_index[0], pad_ids])
    dst = jnp.concatenate([edge_index[1], pad_ids])
    src2d = src.reshape(NW, 128)
    dst2d = dst.reshape(NW, 128)
    x_pad = jnp.pad(in_feat, ((0, NPAD - N), (0, 0)))
    zeros2d = jnp.zeros((NPAD, F), jnp.float32)
    zeros1d = jnp.zeros((NPAD,), jnp.float32)

    degp = _k1_degrees(src, dst, zeros1d)
    norms = _k2a_norms(degp.reshape(32, NPB, 128))
    ns_col = norms[0].reshape(NPAD, 1)
    nd_col = norms[1].reshape(NPAD, 1)
    nd_flat = norms[1].reshape(NPAD)
    cp = _k3c_cweights(src, dst, nd_flat, zeros1d)
    xn = _k2b_scale(x_pad, ns_col)
    aggp = _k3_message_pass(src2d, dst2d, xn, zeros2d)
    return _k4_final(aggp, cp.reshape(32, NPAD, 1), ns_col, nd_col, W1,
                     b1.reshape(1, F), W2, b2.reshape(1, C))


# speedup vs baseline: 1.3111x; 1.3111x over previous
import dataclasses
import functools

import jax
import jax.numpy as jnp
from jax import lax
from jax.experimental import pallas as pl
from jax.experimental.pallas import tpu as pltpu
from jax.experimental.pallas import tpu_sc as plsc

N = 10000
E = 320000
NPAD = 10240
EPAD = 327680
NW = 2560
F = 128
C = 64
NPB = NPAD // 128

_mesh = plsc.VectorSubcoreMesh(core_axis_name="c", subcore_axis_name="s")

_sc_params = pltpu.CompilerParams()
if "needs_layout_passes" in pltpu.CompilerParams.__dataclass_fields__:
    _sc_params = dataclasses.replace(_sc_params, needs_layout_passes=False)


@jax.jit
def _k1_degrees(src_flat, dst_flat, zeros1d):
    @functools.partial(
        pl.kernel,
        out_type=jax.ShapeDtypeStruct((2, 16, NPAD), jnp.float32),
        mesh=_mesh,
        scratch_types=[
            pltpu.VMEM((EPAD // 16,), jnp.int32),
            pltpu.VMEM((NPAD,), jnp.float32),
        ],
        compiler_params=_sc_params,
    )
    def k1(src_hbm, dst_hbm, z_hbm, degp_out, idx_ref, deg_ref):
        cid = lax.axis_index("c")
        sid = lax.axis_index("s")
        chunk = EPAD // 16
        base = sid * chunk
        pltpu.sync_copy(z_hbm, deg_ref)

        @pl.when(cid == 0)
        def _():
            pltpu.sync_copy(src_hbm.at[pl.ds(base, chunk)], idx_ref)

        @pl.when(cid == 1)
        def _():
            pltpu.sync_copy(dst_hbm.at[pl.ds(base, chunk)], idx_ref)

        ones = jnp.full((16,), 1.0, jnp.float32)

        @pl.loop(0, chunk, step=16, unroll=8)
        def _(i):
            iv = idx_ref[pl.ds(i, 16)]
            plsc.addupdate_scatter(deg_ref, [iv], ones)

        pltpu.sync_copy(deg_ref, degp_out.at[cid].at[sid])

    return k1(src_flat, dst_flat, zeros1d)


@jax.jit
def _k2a_norms(degp):
    def body(degp_ref, norms_ref):
        d_out = jnp.sum(degp_ref[0:16], axis=0)
        d_in = jnp.sum(degp_ref[16:32], axis=0)
        row = lax.broadcasted_iota(jnp.int32, (NPB, 128), 0)
        col = lax.broadcasted_iota(jnp.int32, (NPB, 128), 1)
        real = (row * 128 + col) < N
        ns = jnp.where(real, lax.rsqrt(jnp.maximum(d_out, 1.0)), 0.0)
        nd = jnp.where(real, lax.rsqrt(jnp.maximum(d_in, 1.0)), 0.0)
        norms_ref[0] = ns
        norms_ref[1] = nd

    return pl.pallas_call(
        body,
        out_shape=jax.ShapeDtypeStruct((2, NPB, 128), jnp.float32),
    )(degp)


@jax.jit
def _k2b_scale(x_pad, ns_col):
    def body(x_ref, ns_ref, xn_ref):
        xn_ref[...] = x_ref[...] * ns_ref[...]

    return pl.pallas_call(
        body,
        grid=(NPAD // 512,),
        in_specs=[
            pl.BlockSpec((512, F), lambda i: (i, 0)),
            pl.BlockSpec((512, 1), lambda i: (i, 0)),
        ],
        out_specs=pl.BlockSpec((512, F), lambda i: (i, 0)),
        out_shape=jax.ShapeDtypeStruct((NPAD, F), jnp.float32),
    )(x_pad, ns_col)


@jax.jit
def _k3c_cweights(src_flat, dst_flat, nd_flat, zeros1d):
    chunk = EPAD // 32

    @functools.partial(
        pl.kernel,
        out_type=jax.ShapeDtypeStruct((2, 16, NPAD), jnp.float32),
        mesh=_mesh,
        scratch_types=[
            pltpu.VMEM((chunk,), jnp.int32),
            pltpu.VMEM((chunk,), jnp.int32),
            pltpu.VMEM((NPAD,), jnp.float32),
            pltpu.VMEM((NPAD,), jnp.float32),
        ],
        compiler_params=_sc_params,
    )
    def k3c(src_hbm, dst_hbm, nd_hbm, z_hbm, c_out,
            sidx, didx, nd_ref, c_ref):
        cid = lax.axis_index("c")
        sid = lax.axis_index("s")
        base = (cid * 16 + sid) * chunk
        pltpu.sync_copy(z_hbm, c_ref)
        pltpu.sync_copy(nd_hbm, nd_ref)
        pltpu.sync_copy(src_hbm.at[pl.ds(base, chunk)], sidx)
        pltpu.sync_copy(dst_hbm.at[pl.ds(base, chunk)], didx)

        @pl.loop(0, chunk, step=16, unroll=8)
        def _(i):
            dv = didx[pl.ds(i, 16)]
            sv = sidx[pl.ds(i, 16)]
            ndv = plsc.load_gather(nd_ref, [dv])
            plsc.addupdate_scatter(c_ref, [sv], ndv)

        pltpu.sync_copy(c_ref, c_out.at[cid].at[sid])

    return k3c(src_flat, dst_flat, nd_flat, zeros1d)


@jax.jit
def _k3_message_pass(src2d, dst2d, xn, zeros2d):
    wlen = 64
    nwin = EPAD // wlen
    wpw = nwin // 32
    half = wpw // 4

    @functools.partial(
        pl.kernel,
        out_type=jax.ShapeDtypeStruct((2, NPAD, F), jnp.float32),
        mesh=_mesh,
        scratch_types=[
            pltpu.VMEM((half, wlen), jnp.int32),
            pltpu.VMEM((half, wlen), jnp.int32),
            pltpu.VMEM((wlen, F), jnp.float32),
            pltpu.VMEM((wlen, F), jnp.float32),
            pltpu.VMEM((wlen, F), jnp.float32),
            pltpu.VMEM((wlen, F), jnp.float32),
            pltpu.SemaphoreType.DMA,
            pltpu.SemaphoreType.DMA,
            pltpu.SemaphoreType.DMA,
            pltpu.SemaphoreType.DMA,
            pltpu.VMEM_SHARED((NPAD, F), jnp.float32),
        ],
        compiler_params=_sc_params,
    )
    def k3(src_hbm, dst_hbm, xn_hbm, z2_hbm, agg_out,
           sidx, didx, r0, r1, r2, r3, s0, s1, s2, s3, agg_sp):
        cid = lax.axis_index("c")
        sid = lax.axis_index("s")
        wid = cid * 16 + sid
        rows = (r0, r1, r2, r3)
        sems = (s0, s1, s2, s3)

        pltpu.sync_copy(z2_hbm.at[pl.ds(sid * 640, 640)],
                        agg_sp.at[pl.ds(sid * 640, 640)])
        plsc.subcore_barrier()

        for h in range(4):
            base = wid * wpw + h * half
            pltpu.sync_copy(src_hbm.at[pl.ds(base, half)], sidx)
            pltpu.sync_copy(dst_hbm.at[pl.ds(base, half)], didx)
            for b in range(4):
                pltpu.async_copy(xn_hbm.at[sidx.at[b]], rows[b], sems[b])

            @pl.loop(0, half, step=4)
            def _(k):
                for b in range(4):
                    w = k + b
                    pltpu.make_async_copy(
                        xn_hbm.at[sidx.at[w]], rows[b], sems[b]).wait()
                    pltpu.sync_copy(rows[b], agg_sp.at[didx.at[w]],
                                    add=True)

                    @pl.when(w + 4 < half)
                    def _():
                        pltpu.async_copy(
                            xn_hbm.at[sidx.at[w + 4]], rows[b], sems[b])

        plsc.subcore_barrier()
        pltpu.sync_copy(agg_sp.at[pl.ds(sid * 640, 640)],
                        agg_out.at[cid].at[pl.ds(sid * 640, 640)])

    return k3(src2d, dst2d, xn, zeros2d)


@jax.jit
def _k3b_weights(cp, norms):
    def body(cp_ref, norms_ref, w_ref):
        c = jnp.sum(cp_ref[...], axis=0)
        w_ref[...] = c * norms_ref[0] * (1.0 / N)

    return pl.pallas_call(
        body,
        out_shape=jax.ShapeDtypeStruct((NPB, 128), jnp.float32),
    )(cp, norms)


@jax.jit
def _k4_final(aggp, nd_col, w_col, W1, b1, W2, b2):
    nblk = NPAD // 512

    def body(aggp_ref, nd_ref, w_ref, W1_ref, b1_ref, W2_ref, b2_ref,
             out_ref, acc_ref):
        i = pl.program_id(0)
        agg = (aggp_ref[0] + aggp_ref[1]) * nd_ref[...]
        z = jnp.dot(agg, W1_ref[...], preferred_element_type=jnp.float32)
        h = jnp.maximum(z + b1_ref[...], 0.0)
        pb = jnp.sum(h * w_ref[...], axis=0, keepdims=True)

        @pl.when(i == 0)
        def _():
            acc_ref[...] = jnp.zeros_like(acc_ref)

        acc_ref[...] += pb

        @pl.when(i == nblk - 1)
        def _():
            out_ref[...] = (
                jnp.dot(acc_ref[...], W2_ref[...],
                        preferred_element_type=jnp.float32)
                + b2_ref[...]
            )

    return pl.pallas_call(
        body,
        grid=(nblk,),
        in_specs=[
            pl.BlockSpec((2, 512, F), lambda i: (0, i, 0)),
            pl.BlockSpec((512, 1), lambda i: (i, 0)),
            pl.BlockSpec((512, 1), lambda i: (i, 0)),
            pl.BlockSpec((F, F), lambda i: (0, 0)),
            pl.BlockSpec((1, F), lambda i: (0, 0)),
            pl.BlockSpec((F, C), lambda i: (0, 0)),
            pl.BlockSpec((1, C), lambda i: (0, 0)),
        ],
        out_specs=pl.BlockSpec((1, C), lambda i: (0, 0)),
        out_shape=jax.ShapeDtypeStruct((1, C), jnp.float32),
        scratch_shapes=[pltpu.VMEM((1, F), jnp.float32)],
    )(aggp, nd_col, w_col, W1, b1, W2, b2)


def kernel(in_feat, edge_index, W1, b1, W2, b2):
    pad_ids = (N + (jnp.arange(EPAD - E, dtype=jnp.int32) % (NPAD - N)))
    src = jnp.concatenate([edge_index[0], pad_ids])
    dst = jnp.concatenate([edge_index[1], pad_ids])
    src2d = src.reshape(EPAD // 64, 64)
    dst2d = dst.reshape(EPAD // 64, 64)
    x_pad = jnp.pad(in_feat, ((0, NPAD - N), (0, 0)))
    zeros2d = jnp.zeros((NPAD, F), jnp.float32)
    zeros1d = jnp.zeros((NPAD,), jnp.float32)

    degp = _k1_degrees(src, dst, zeros1d)
    norms = _k2a_norms(degp.reshape(32, NPB, 128))
    ns_col = norms[0].reshape(NPAD, 1)
    nd_col = norms[1].reshape(NPAD, 1)
    nd_flat = norms[1].reshape(NPAD)
    cp = _k3c_cweights(src, dst, nd_flat, zeros1d)
    xn = _k2b_scale(x_pad, ns_col)
    aggp = _k3_message_pass(src2d, dst2d, xn, zeros2d)
    w_col = _k3b_weights(cp.reshape(32, NPB, 128), norms).reshape(NPAD, 1)
    return _k4_final(aggp, nd_col, w_col, W1,
                     b1.reshape(1, F), W2, b2.reshape(1, C))


# speedup vs baseline: 1.3547x; 1.0332x over previous
import dataclasses
import functools

import jax
import jax.numpy as jnp
from jax import lax
from jax.experimental import pallas as pl
from jax.experimental.pallas import tpu as pltpu
from jax.experimental.pallas import tpu_sc as plsc

N = 10000
E = 320000
NPAD = 10240
EPAD = 327680
NW = 2560
F = 128
C = 64
NPB = NPAD // 128

_mesh = plsc.VectorSubcoreMesh(core_axis_name="c", subcore_axis_name="s")

_sc_params = pltpu.CompilerParams()
if "needs_layout_passes" in pltpu.CompilerParams.__dataclass_fields__:
    _sc_params = dataclasses.replace(_sc_params, needs_layout_passes=False)


@jax.jit
def _k1_degrees(src_flat, dst_flat, zeros1d):
    @functools.partial(
        pl.kernel,
        out_type=jax.ShapeDtypeStruct((2, 16, NPAD), jnp.float32),
        mesh=_mesh,
        scratch_types=[
            pltpu.VMEM((EPAD // 16,), jnp.int32),
            pltpu.VMEM((NPAD,), jnp.float32),
        ],
        compiler_params=_sc_params,
    )
    def k1(src_hbm, dst_hbm, z_hbm, degp_out, idx_ref, deg_ref):
        cid = lax.axis_index("c")
        sid = lax.axis_index("s")
        chunk = EPAD // 16
        base = sid * chunk
        pltpu.sync_copy(z_hbm, deg_ref)

        @pl.when(cid == 0)
        def _():
            pltpu.sync_copy(src_hbm.at[pl.ds(base, chunk)], idx_ref)

        @pl.when(cid == 1)
        def _():
            pltpu.sync_copy(dst_hbm.at[pl.ds(base, chunk)], idx_ref)

        ones = jnp.full((16,), 1.0, jnp.float32)

        @pl.loop(0, chunk, step=16, unroll=8)
        def _(i):
            iv = idx_ref[pl.ds(i, 16)]
            plsc.addupdate_scatter(deg_ref, [iv], ones)

        pltpu.sync_copy(deg_ref, degp_out.at[cid].at[sid])

    return k1(src_flat, dst_flat, zeros1d)


@jax.jit
def _k2a_norms(degp):
    def body(degp_ref, norms_ref):
        d_out = jnp.sum(degp_ref[0:16], axis=0)
        d_in = jnp.sum(degp_ref[16:32], axis=0)
        row = lax.broadcasted_iota(jnp.int32, (NPB, 128), 0)
        col = lax.broadcasted_iota(jnp.int32, (NPB, 128), 1)
        real = (row * 128 + col) < N
        ns = jnp.where(real, lax.rsqrt(jnp.maximum(d_out, 1.0)), 0.0)
        nd = jnp.where(real, lax.rsqrt(jnp.maximum(d_in, 1.0)), 0.0)
        norms_ref[0] = ns
        norms_ref[1] = nd

    return pl.pallas_call(
        body,
        out_shape=jax.ShapeDtypeStruct((2, NPB, 128), jnp.float32),
    )(degp)


@jax.jit
def _k2b_scale(x_pad, ns_col):
    def body(x_ref, ns_ref, xn_ref):
        xn_ref[...] = x_ref[...] * ns_ref[...]

    return pl.pallas_call(
        body,
        grid=(NPAD // 512,),
        in_specs=[
            pl.BlockSpec((512, F), lambda i: (i, 0)),
            pl.BlockSpec((512, 1), lambda i: (i, 0)),
        ],
        out_specs=pl.BlockSpec((512, F), lambda i: (i, 0)),
        out_shape=jax.ShapeDtypeStruct((NPAD, F), jnp.float32),
    )(x_pad, ns_col)


@jax.jit
def _k3c_cweights(src_flat, dst_flat, nd_flat, zeros1d):
    chunk = EPAD // 32

    @functools.partial(
        pl.kernel,
        out_type=jax.ShapeDtypeStruct((2, 16, NPAD), jnp.float32),
        mesh=_mesh,
        scratch_types=[
            pltpu.VMEM((chunk,), jnp.int32),
            pltpu.VMEM((chunk,), jnp.int32),
            pltpu.VMEM((NPAD,), jnp.float32),
            pltpu.VMEM((NPAD,), jnp.float32),
        ],
        compiler_params=_sc_params,
    )
    def k3c(src_hbm, dst_hbm, nd_hbm, z_hbm, c_out,
            sidx, didx, nd_ref, c_ref):
        cid = lax.axis_index("c")
        sid = lax.axis_index("s")
        base = (cid * 16 + sid) * chunk
        pltpu.sync_copy(z_hbm, c_ref)
        pltpu.sync_copy(nd_hbm, nd_ref)
        pltpu.sync_copy(src_hbm.at[pl.ds(base, chunk)], sidx)
        pltpu.sync_copy(dst_hbm.at[pl.ds(base, chunk)], didx)

        @pl.loop(0, chunk, step=16, unroll=8)
        def _(i):
            dv = didx[pl.ds(i, 16)]
            sv = sidx[pl.ds(i, 16)]
            ndv = plsc.load_gather(nd_ref, [dv])
            plsc.addupdate_scatter(c_ref, [sv], ndv)

        pltpu.sync_copy(c_ref, c_out.at[cid].at[sid])

    return k3c(src_flat, dst_flat, nd_flat, zeros1d)


@jax.jit
def _k3_message_pass(src2d, dst2d, xn):
    wpw = NW // 32
    half = wpw // 2

    @functools.partial(
        pl.kernel,
        out_type=jax.ShapeDtypeStruct((2, NPAD, F), jnp.float32),
        mesh=_mesh,
        scratch_types=[
            pltpu.VMEM((half, 128), jnp.int32),
            pltpu.VMEM((half, 128), jnp.int32),
            pltpu.VMEM((128, F), jnp.float32),
            pltpu.VMEM((128, F), jnp.float32),
            pltpu.SemaphoreType.DMA,
            pltpu.SemaphoreType.DMA,
            pltpu.VMEM_SHARED((NPAD, F), jnp.float32),
        ],
        compiler_params=_sc_params,
    )
    def k3(src_hbm, dst_hbm, xn_hbm,
           agg_out, sidx, didx, rowsA, rowsB, semA, semB, agg_sp):
        cid = lax.axis_index("c")
        sid = lax.axis_index("s")
        wid = cid * 16 + sid
        base = wid * wpw

        zv = jnp.zeros((16,), jnp.float32)

        @pl.loop(0, 128)
        def _(r):
            @pl.loop(0, F, step=16)
            def _(j):
                rowsA.at[r][pl.ds(j, 16)] = zv

        @pl.loop(0, 640, step=128)
        def _(r):
            pltpu.sync_copy(rowsA, agg_sp.at[pl.ds(sid * 640 + r, 128)])

        plsc.subcore_barrier()

        for h in range(2):
            pltpu.sync_copy(src_hbm.at[pl.ds(base + h * half, half)], sidx)
            pltpu.sync_copy(dst_hbm.at[pl.ds(base + h * half, half)], didx)
            pltpu.async_copy(xn_hbm.at[sidx.at[0]], rowsA, semA)

            @pl.loop(0, half, step=2)
            def _(k):
                pltpu.async_copy(xn_hbm.at[sidx.at[k + 1]], rowsB, semB)
                pltpu.make_async_copy(
                    xn_hbm.at[sidx.at[k]], rowsA, semA).wait()
                pltpu.sync_copy(rowsA, agg_sp.at[didx.at[k]], add=True)

                @pl.when(k + 2 < half)
                def _():
                    pltpu.async_copy(
                        xn_hbm.at[sidx.at[k + 2]], rowsA, semA)

                pltpu.make_async_copy(
                    xn_hbm.at[sidx.at[k + 1]], rowsB, semB).wait()
                pltpu.sync_copy(rowsB, agg_sp.at[didx.at[k + 1]], add=True)

        plsc.subcore_barrier()
        pltpu.sync_copy(agg_sp.at[pl.ds(sid * 640, 640)],
                        agg_out.at[cid].at[pl.ds(sid * 640, 640)])

    return k3(src2d, dst2d, xn)


@jax.jit
def _k3b_weights(cp, norms):
    def body(cp_ref, norms_ref, w_ref):
        c = jnp.sum(cp_ref[...], axis=0)
        w_ref[...] = c * norms_ref[0] * (1.0 / N)

    return pl.pallas_call(
        body,
        out_shape=jax.ShapeDtypeStruct((NPB, 128), jnp.float32),
    )(cp, norms)


@jax.jit
def _k4_final(aggp, nd_col, w_col, W1, b1, W2, b2):
    nblk = NPAD // 512

    def body(aggp_ref, nd_ref, w_ref, W1_ref, b1_ref, W2_ref, b2_ref,
             out_ref, acc_ref):
        i = pl.program_id(0)
        agg = (aggp_ref[0] + aggp_ref[1]) * nd_ref[...]
        z = jnp.dot(agg, W1_ref[...], preferred_element_type=jnp.float32)
        h = jnp.maximum(z + b1_ref[...], 0.0)
        pb = jnp.sum(h * w_ref[...], axis=0, keepdims=True)

        @pl.when(i == 0)
        def _():
            acc_ref[...] = jnp.zeros_like(acc_ref)

        acc_ref[...] += pb

        @pl.when(i == nblk - 1)
        def _():
            out_ref[...] = (
                jnp.dot(acc_ref[...], W2_ref[...],
                        preferred_element_type=jnp.float32)
                + b2_ref[...]
            )

    return pl.pallas_call(
        body,
        grid=(nblk,),
        in_specs=[
            pl.BlockSpec((2, 512, F), lambda i: (0, i, 0)),
            pl.BlockSpec((512, 1), lambda i: (i, 0)),
            pl.BlockSpec((512, 1), lambda i: (i, 0)),
            pl.BlockSpec((F, F), lambda i: (0, 0)),
            pl.BlockSpec((1, F), lambda i: (0, 0)),
            pl.BlockSpec((F, C), lambda i: (0, 0)),
            pl.BlockSpec((1, C), lambda i: (0, 0)),
        ],
        out_specs=pl.BlockSpec((1, C), lambda i: (0, 0)),
        out_shape=jax.ShapeDtypeStruct((1, C), jnp.float32),
        scratch_shapes=[pltpu.VMEM((1, F), jnp.float32)],
    )(aggp, nd_col, w_col, W1, b1, W2, b2)


def kernel(in_feat, edge_index, W1, b1, W2, b2):
    pad_ids = (N + (jnp.arange(EPAD - E, dtype=jnp.int32) % (NPAD - N)))
    src = jnp.concatenate([edge_index[0], pad_ids])
    dst = jnp.concatenate([edge_index[1], pad_ids])
    src2d = src.reshape(NW, 128)
    dst2d = dst.reshape(NW, 128)
    x_pad = jnp.pad(in_feat, ((0, NPAD - N), (0, 0)))
    zeros1d = jnp.zeros((NPAD,), jnp.float32)

    degp = _k1_degrees(src, dst, zeros1d)
    norms = _k2a_norms(degp.reshape(32, NPB, 128))
    ns_col = norms[0].reshape(NPAD, 1)
    nd_col = norms[1].reshape(NPAD, 1)
    nd_flat = norms[1].reshape(NPAD)
    cp = _k3c_cweights(src, dst, nd_flat, zeros1d)
    xn = _k2b_scale(x_pad, ns_col)
    aggp = _k3_message_pass(src2d, dst2d, xn)
    w_col = _k3b_weights(cp.reshape(32, NPB, 128), norms).reshape(NPAD, 1)
    return _k4_final(aggp, nd_col, w_col, W1,
                     b1.reshape(1, F), W2, b2.reshape(1, C))


# speedup vs baseline: 1.3866x; 1.0235x over previous
import dataclasses
import functools

import jax
import jax.numpy as jnp
from jax import lax
from jax.experimental import pallas as pl
from jax.experimental.pallas import tpu as pltpu
from jax.experimental.pallas import tpu_sc as plsc

N = 10000
E = 320000
NPAD = 10240
EPAD = 327680
NW = 2560
F = 128
C = 64
NPB = NPAD // 128

_mesh = plsc.VectorSubcoreMesh(core_axis_name="c", subcore_axis_name="s")

_sc_params = pltpu.CompilerParams()
if "needs_layout_passes" in pltpu.CompilerParams.__dataclass_fields__:
    _sc_params = dataclasses.replace(_sc_params, needs_layout_passes=False)


@jax.jit
def _k1_degrees(src_flat, dst_flat):
    @functools.partial(
        pl.kernel,
        out_type=jax.ShapeDtypeStruct((2, 16, NPAD), jnp.float32),
        mesh=_mesh,
        scratch_types=[
            pltpu.VMEM((EPAD // 16,), jnp.int32),
            pltpu.VMEM((NPAD,), jnp.float32),
        ],
        compiler_params=_sc_params,
    )
    def k1(src_hbm, dst_hbm, degp_out, idx_ref, deg_ref):
        cid = lax.axis_index("c")
        sid = lax.axis_index("s")
        chunk = EPAD // 16
        base = sid * chunk
        zv = jnp.zeros((16,), jnp.float32)

        @pl.loop(0, NPAD, step=16, unroll=8)
        def _(i):
            deg_ref[pl.ds(i, 16)] = zv

        @pl.when(cid == 0)
        def _():
            pltpu.sync_copy(src_hbm.at[pl.ds(base, chunk)], idx_ref)

        @pl.when(cid == 1)
        def _():
            pltpu.sync_copy(dst_hbm.at[pl.ds(base, chunk)], idx_ref)

        ones = jnp.full((16,), 1.0, jnp.float32)

        @pl.loop(0, chunk, step=16, unroll=8)
        def _(i):
            iv = idx_ref[pl.ds(i, 16)]
            plsc.addupdate_scatter(deg_ref, [iv], ones)

        pltpu.sync_copy(deg_ref, degp_out.at[cid].at[sid])

    return k1(src_flat, dst_flat)


@jax.jit
def _k2a_norms(degp):
    def body(degp_ref, norms_ref):
        d_out = jnp.sum(degp_ref[0:16], axis=0)
        d_in = jnp.sum(degp_ref[16:32], axis=0)
        row = lax.broadcasted_iota(jnp.int32, (NPB, 128), 0)
        col = lax.broadcasted_iota(jnp.int32, (NPB, 128), 1)
        real = (row * 128 + col) < N
        ns = jnp.where(real, lax.rsqrt(jnp.maximum(d_out, 1.0)), 0.0)
        nd = jnp.where(real, lax.rsqrt(jnp.maximum(d_in, 1.0)), 0.0)
        norms_ref[0] = ns
        norms_ref[1] = nd

    return pl.pallas_call(
        body,
        out_shape=jax.ShapeDtypeStruct((2, NPB, 128), jnp.float32),
    )(degp)


@jax.jit
def _k2b_scale(x_pad, ns_col):
    def body(x_ref, ns_ref, xn_ref):
        xn_ref[...] = x_ref[...] * ns_ref[...]

    return pl.pallas_call(
        body,
        grid=(NPAD // 512,),
        in_specs=[
            pl.BlockSpec((512, F), lambda i: (i, 0)),
            pl.BlockSpec((512, 1), lambda i: (i, 0)),
        ],
        out_specs=pl.BlockSpec((512, F), lambda i: (i, 0)),
        out_shape=jax.ShapeDtypeStruct((NPAD, F), jnp.float32),
    )(x_pad, ns_col)


@jax.jit
def _k3c_cweights(src_flat, dst_flat, nd_flat):
    chunk = EPAD // 32

    @functools.partial(
        pl.kernel,
        out_type=jax.ShapeDtypeStruct((2, 16, NPAD), jnp.float32),
        mesh=_mesh,
        scratch_types=[
            pltpu.VMEM((chunk,), jnp.int32),
            pltpu.VMEM((chunk,), jnp.int32),
            pltpu.VMEM((NPAD,), jnp.float32),
            pltpu.VMEM((NPAD,), jnp.float32),
        ],
        compiler_params=_sc_params,
    )
    def k3c(src_hbm, dst_hbm, nd_hbm, c_out,
            sidx, didx, nd_ref, c_ref):
        cid = lax.axis_index("c")
        sid = lax.axis_index("s")
        base = (cid * 16 + sid) * chunk
        zv = jnp.zeros((16,), jnp.float32)

        @pl.loop(0, NPAD, step=16, unroll=8)
        def _(i):
            c_ref[pl.ds(i, 16)] = zv

        pltpu.sync_copy(nd_hbm, nd_ref)
        pltpu.sync_copy(src_hbm.at[pl.ds(base, chunk)], sidx)
        pltpu.sync_copy(dst_hbm.at[pl.ds(base, chunk)], didx)

        @pl.loop(0, chunk, step=16, unroll=8)
        def _(i):
            dv = didx[pl.ds(i, 16)]
            sv = sidx[pl.ds(i, 16)]
            ndv = plsc.load_gather(nd_ref, [dv])
            plsc.addupdate_scatter(c_ref, [sv], ndv)

        pltpu.sync_copy(c_ref, c_out.at[cid].at[sid])

    return k3c(src_flat, dst_flat, nd_flat)


@jax.jit
def _k3_message_pass(src2d, dst2d, xn):
    wpw = NW // 32
    half = wpw // 2

    @functools.partial(
        pl.kernel,
        out_type=jax.ShapeDtypeStruct((2, NPAD, F), jnp.float32),
        mesh=_mesh,
        scratch_types=[
            pltpu.VMEM((half, 128), jnp.int32),
            pltpu.VMEM((half, 128), jnp.int32),
            pltpu.VMEM((128, F), jnp.float32),
            pltpu.VMEM((128, F), jnp.float32),
            pltpu.SemaphoreType.DMA,
            pltpu.SemaphoreType.DMA,
            pltpu.VMEM_SHARED((NPAD, F), jnp.float32),
        ],
        compiler_params=_sc_params,
    )
    def k3(src_hbm, dst_hbm, xn_hbm,
           agg_out, sidx, didx, rowsA, rowsB, semA, semB, agg_sp):
        cid = lax.axis_index("c")
        sid = lax.axis_index("s")
        wid = cid * 16 + sid
        base = wid * wpw

        zv = jnp.zeros((16,), jnp.float32)

        @pl.loop(0, 128)
        def _(r):
            @pl.loop(0, F, step=16)
            def _(j):
                rowsA.at[r][pl.ds(j, 16)] = zv

        @pl.loop(0, 640, step=128)
        def _(r):
            pltpu.sync_copy(rowsA, agg_sp.at[pl.ds(sid * 640 + r, 128)])

        plsc.subcore_barrier()

        for h in range(2):
            pltpu.sync_copy(src_hbm.at[pl.ds(base + h * half, half)], sidx)
            pltpu.sync_copy(dst_hbm.at[pl.ds(base + h * half, half)], didx)
            pltpu.async_copy(xn_hbm.at[sidx.at[0]], rowsA, semA)

            @pl.loop(0, half, step=2)
            def _(k):
                pltpu.async_copy(xn_hbm.at[sidx.at[k + 1]], rowsB, semB)
                pltpu.make_async_copy(
                    xn_hbm.at[sidx.at[k]], rowsA, semA).wait()
                pltpu.sync_copy(rowsA, agg_sp.at[didx.at[k]], add=True)

                @pl.when(k + 2 < half)
                def _():
                    pltpu.async_copy(
                        xn_hbm.at[sidx.at[k + 2]], rowsA, semA)

                pltpu.make_async_copy(
                    xn_hbm.at[sidx.at[k + 1]], rowsB, semB).wait()
                pltpu.sync_copy(rowsB, agg_sp.at[didx.at[k + 1]], add=True)

        plsc.subcore_barrier()
        pltpu.sync_copy(agg_sp.at[pl.ds(sid * 640, 640)],
                        agg_out.at[cid].at[pl.ds(sid * 640, 640)])

    return k3(src2d, dst2d, xn)


@jax.jit
def _k3b_weights(cp, norms):
    def body(cp_ref, norms_ref, w_ref):
        c = jnp.sum(cp_ref[...], axis=0)
        w_ref[...] = c * norms_ref[0] * (1.0 / N)

    return pl.pallas_call(
        body,
        out_shape=jax.ShapeDtypeStruct((NPB, 128), jnp.float32),
    )(cp, norms)


@jax.jit
def _k4_final(aggp, nd_col, w_col, W1, b1, W2, b2):
    nblk = NPAD // 512

    def body(aggp_ref, nd_ref, w_ref, W1_ref, b1_ref, W2_ref, b2_ref,
             out_ref, acc_ref):
        i = pl.program_id(0)
        agg = (aggp_ref[0] + aggp_ref[1]) * nd_ref[...]
        z = jnp.dot(agg, W1_ref[...], preferred_element_type=jnp.float32)
        h = jnp.maximum(z + b1_ref[...], 0.0)
        pb = jnp.sum(h * w_ref[...], axis=0, keepdims=True)

        @pl.when(i == 0)
        def _():
            acc_ref[...] = jnp.zeros_like(acc_ref)

        acc_ref[...] += pb

        @pl.when(i == nblk - 1)
        def _():
            out_ref[...] = (
                jnp.dot(acc_ref[...], W2_ref[...],
                        preferred_element_type=jnp.float32)
                + b2_ref[...]
            )

    return pl.pallas_call(
        body,
        grid=(nblk,),
        in_specs=[
            pl.BlockSpec((2, 512, F), lambda i: (0, i, 0)),
            pl.BlockSpec((512, 1), lambda i: (i, 0)),
            pl.BlockSpec((512, 1), lambda i: (i, 0)),
            pl.BlockSpec((F, F), lambda i: (0, 0)),
            pl.BlockSpec((1, F), lambda i: (0, 0)),
            pl.BlockSpec((F, C), lambda i: (0, 0)),
            pl.BlockSpec((1, C), lambda i: (0, 0)),
        ],
        out_specs=pl.BlockSpec((1, C), lambda i: (0, 0)),
        out_shape=jax.ShapeDtypeStruct((1, C), jnp.float32),
        scratch_shapes=[pltpu.VMEM((1, F), jnp.float32)],
    )(aggp, nd_col, w_col, W1, b1, W2, b2)


def kernel(in_feat, edge_index, W1, b1, W2, b2):
    pad_ids = (N + (jnp.arange(EPAD - E, dtype=jnp.int32) % (NPAD - N)))
    src = jnp.concatenate([edge_index[0], pad_ids])
    dst = jnp.concatenate([edge_index[1], pad_ids])
    src2d = src.reshape(NW, 128)
    dst2d = dst.reshape(NW, 128)
    x_pad = jnp.pad(in_feat, ((0, NPAD - N), (0, 0)))

    degp = _k1_degrees(src, dst)
    norms = _k2a_norms(degp.reshape(32, NPB, 128))
    ns_col = norms[0].reshape(NPAD, 1)
    nd_col = norms[1].reshape(NPAD, 1)
    nd_flat = norms[1].reshape(NPAD)
    cp = _k3c_cweights(src, dst, nd_flat)
    xn = _k2b_scale(x_pad, ns_col)
    aggp = _k3_message_pass(src2d, dst2d, xn)
    w_col = _k3b_weights(cp.reshape(32, NPB, 128), norms).reshape(NPAD, 1)
    return _k4_final(aggp, nd_col, w_col, W1,
                     b1.reshape(1, F), W2, b2.reshape(1, C))
